# baseline (speedup 1.0000x reference)
import dataclasses
import functools
import math

import jax
import jax.numpy as jnp
from jax import lax
from jax.experimental import pallas as pl
from jax.experimental.pallas import tpu as pltpu
from jax.experimental.pallas import tpu_sc as plsc

D_MODEL = 128
SCALE = math.sqrt(float(D_MODEL))

NUM_CORES = 2
NUM_SUBCORES = 16
NUM_LANES = 16
NW = NUM_CORES * NUM_SUBCORES

CHUNK = 128
HALF = CHUNK // 2
K_SLICES = 5


def _build_sc_pack(n_chunks: int):
    mesh = plsc.VectorSubcoreMesh(core_axis_name="c", subcore_axis_name="s")
    total_pack_rows = NW * n_chunks * HALF

    DEPTH = 3
    n_main = (n_chunks // DEPTH) * DEPTH

    cp = pltpu.CompilerParams()
    if "needs_layout_passes" in pltpu.CompilerParams.__dataclass_fields__:
        cp = dataclasses.replace(cp, needs_layout_passes=False)

    @functools.partial(
        pl.kernel,
        mesh=mesh,
        compiler_params=cp,
        out_type=jax.ShapeDtypeStruct((total_pack_rows, D_MODEL), jnp.int32),
        scratch_types=[
            pltpu.VMEM((n_chunks * CHUNK,), jnp.int32),
        ]
        + [pltpu.VMEM((CHUNK, D_MODEL), jnp.float32)] * DEPTH
        + [pltpu.VMEM((HALF, D_MODEL), jnp.int32)] * DEPTH
        + [pltpu.SemaphoreType.DMA] * (2 * DEPTH),
    )
    def k(table_hbm, idx_hbm, out_hbm, idx_v, *bufs_sems):
        ibufs = bufs_sems[0:DEPTH]
        obufs = bufs_sems[DEPTH:2 * DEPTH]
        isems = bufs_sems[2 * DEPTH:3 * DEPTH]
        osems = bufs_sems[3 * DEPTH:4 * DEPTH]

        wid = lax.axis_index("s") * NUM_CORES + lax.axis_index("c")
        base = wid * (n_chunks * HALF)
        pltpu.sync_copy(idx_hbm.at[pl.ds(wid * n_chunks * CHUNK,
                                         n_chunks * CHUNK)], idx_v)

        def start_gather(j, s):
            idx = idx_v.at[pl.ds(j * CHUNK, CHUNK)]
            pltpu.make_async_copy(table_hbm.at[idx], ibufs[s], isems[s]).start()

        def wait_gather(j, s):
            idx = idx_v.at[pl.ds(j * CHUNK, CHUNK)]
            pltpu.make_async_copy(table_hbm.at[idx], ibufs[s], isems[s]).wait()

        def start_store(j, s):
            dst = out_hbm.at[pl.ds(base + j * HALF, HALF)]
            pltpu.make_async_copy(obufs[s], dst, osems[s]).start()

        def wait_store(s):
            dst = out_hbm.at[pl.ds(base, HALF)]
            pltpu.make_async_copy(obufs[s], dst, osems[s]).wait()

        def pack(s):
            ibuf, obuf = ibufs[s], obufs[s]

            @plsc.parallel_loop(0, HALF, unroll=2)
            def _(r):
                for c in range(D_MODEL // NUM_LANES):
                    sl = pl.ds(c * NUM_LANES, NUM_LANES)
                    u = plsc.bitcast(ibuf.at[r, sl][...], jnp.uint32)
                    v = plsc.bitcast(ibuf.at[r + HALF, sl][...], jnp.uint32)
                    rnd = jnp.uint32(0x8000)
                    lo = lax.shift_right_logical(u + rnd, jnp.uint32(16))
                    hi = (v + rnd) & jnp.uint32(0xFFFF0000)
                    obuf.at[r, sl][...] = plsc.bitcast(hi | lo, jnp.int32)

        for s in range(DEPTH):
            start_gather(s, s)

        @pl.loop(0, n_main, step=DEPTH)
        def _(j):
            for s in range(DEPTH):
                jj = j + s
                wait_gather(jj, s)

                @pl.when(jj >= DEPTH)
                def _():
                    wait_store(s)

                pack(s)
                if n_main + s >= n_chunks:
                    @pl.when(jj + DEPTH < n_chunks)
                    def _():
                        start_gather(jj + DEPTH, s)
                else:
                    start_gather(jj + DEPTH, s)
                start_store(jj, s)

        for j in range(n_main, n_chunks):
            s = j % DEPTH
            wait_gather(j, s)
            wait_store(s)
            pack(s)
            start_store(j, s)

        for s in range(DEPTH):
            wait_store(s)

    return k


def _tc_unpack_body(pk_ref, out_ref):
    w = pk_ref[...]
    lo = jax.lax.bitcast_convert_type(
        lax.shift_left(w, 16), jnp.float32)
    hi = jax.lax.bitcast_convert_type(
        w & jnp.int32(-65536), jnp.float32)
    out_ref[0:HALF, :] = lo * SCALE
    out_ref[HALF:CHUNK, :] = hi * SCALE


def _tc_unpack(pk, prev_out, slice_blocks, block_offset, total_rows):
    out_shape = jax.ShapeDtypeStruct((total_rows, D_MODEL), jnp.float32)
    out_spec = pl.BlockSpec((CHUNK, D_MODEL), lambda g: (block_offset + g, 0))
    pk_spec = pl.BlockSpec((HALF, D_MODEL), lambda g: (g, 0))
    if prev_out is None:
        return pl.pallas_call(
            _tc_unpack_body,
            grid=(slice_blocks,),
            in_specs=[pk_spec],
            out_specs=out_spec,
            out_shape=out_shape,
        )(pk)

    def body(pk_ref, prev_ref, out_ref):
        del prev_ref
        _tc_unpack_body(pk_ref, out_ref)

    return pl.pallas_call(
        body,
        grid=(slice_blocks,),
        in_specs=[pk_spec, pl.BlockSpec(memory_space=pl.ANY)],
        out_specs=out_spec,
        out_shape=out_shape,
        input_output_aliases={1: 0},
    )(pk, prev_out)


def kernel(x, table):
    b, l = x.shape
    vocab, d = table.shape
    assert d == D_MODEL
    n_total = b * l
    assert n_total % (NW * CHUNK * K_SLICES) == 0
    nc_k = n_total // (NW * CHUNK * K_SLICES)
    s_tokens = n_total // K_SLICES
    slice_blocks = s_tokens // CHUNK

    idx = x.reshape(n_total).astype(jnp.int32)
    sc_pack = _build_sc_pack(nc_k)

    pks = []
    for k in range(K_SLICES):
        idx_k = lax.slice(idx, (k * s_tokens,), ((k + 1) * s_tokens,))
        pks.append(sc_pack(table, idx_k))

    out = None
    for k in range(K_SLICES):
        out = _tc_unpack(pks[k], out, slice_blocks, k * slice_blocks, n_total)
    return out.reshape(b, l, d)


# speedup vs baseline: 5.2075x; 5.2075x over previous
import dataclasses
import functools
import math

import jax
import jax.numpy as jnp
from jax import lax
from jax.experimental import pallas as pl
from jax.experimental.pallas import tpu as pltpu
from jax.experimental.pallas import tpu_sc as plsc

D_MODEL = 128
SCALE = math.sqrt(float(D_MODEL))

NUM_CORES = 2
NUM_SUBCORES = 16
NUM_LANES = 16
NW = NUM_CORES * NUM_SUBCORES

CHUNK = 128
HALF = CHUNK // 2
DEPTH = 3

TC_SUB = 8


def _compiler_params():
    cp = pltpu.CompilerParams()
    if "needs_layout_passes" in pltpu.CompilerParams.__dataclass_fields__:
        cp = dataclasses.replace(cp, needs_layout_passes=False)
    return cp


def _sc_pipeline(n_chunks, table_hbm, idx_hbm, out_hbm, idx_v, bufs_sems,
                 compute, store_rows, out_base_fn):
    ibufs = bufs_sems[0:DEPTH]
    obufs = bufs_sems[DEPTH:2 * DEPTH]
    isems = bufs_sems[2 * DEPTH:3 * DEPTH]
    osems = bufs_sems[3 * DEPTH:4 * DEPTH]

    wid = lax.axis_index("s") * NUM_CORES + lax.axis_index("c")
    pltpu.sync_copy(idx_hbm.at[pl.ds(wid * n_chunks * CHUNK,
                                     n_chunks * CHUNK)], idx_v)
    out_base = out_base_fn(wid)

    def start_gather(j, s):
        idx = idx_v.at[pl.ds(j * CHUNK, CHUNK)]
        pltpu.make_async_copy(table_hbm.at[idx], ibufs[s], isems[s]).start()

    def wait_gather(j, s):
        idx = idx_v.at[pl.ds(j * CHUNK, CHUNK)]
        pltpu.make_async_copy(table_hbm.at[idx], ibufs[s], isems[s]).wait()

    def start_store(j, s):
        dst = out_hbm.at[pl.ds(out_base + j * store_rows, store_rows)]
        pltpu.make_async_copy(obufs[s], dst, osems[s]).start()

    def wait_store(s):
        dst = out_hbm.at[pl.ds(out_base, store_rows)]
        pltpu.make_async_copy(obufs[s], dst, osems[s]).wait()

    n_main = (n_chunks // DEPTH) * DEPTH

    for s in range(DEPTH):
        start_gather(s, s)

    @pl.loop(0, n_main, step=DEPTH)
    def _(j):
        for s in range(DEPTH):
            jj = j + s
            wait_gather(jj, s)

            @pl.when(jj >= DEPTH)
            def _():
                wait_store(s)

            compute(ibufs[s], obufs[s])
            if n_main + s >= n_chunks:
                @pl.when(jj + DEPTH < n_chunks)
                def _():
                    start_gather(jj + DEPTH, s)
            else:
                start_gather(jj + DEPTH, s)
            start_store(jj, s)

    for j in range(n_main, n_chunks):
        s = j % DEPTH
        wait_gather(j, s)
        wait_store(s)
        compute(ibufs[s], obufs[s])
        start_store(j, s)

    for s in range(DEPTH):
        wait_store(s)


def _build_sc_pack(n_chunks: int):
    mesh = plsc.VectorSubcoreMesh(core_axis_name="c", subcore_axis_name="s")
    total_pack_rows = NW * n_chunks * HALF

    @functools.partial(
        pl.kernel,
        mesh=mesh,
        compiler_params=_compiler_params(),
        out_type=jax.ShapeDtypeStruct((total_pack_rows, D_MODEL), jnp.int32),
        scratch_types=[
            pltpu.VMEM((n_chunks * CHUNK,), jnp.int32),
        ]
        + [pltpu.VMEM((CHUNK, D_MODEL), jnp.float32)] * DEPTH
        + [pltpu.VMEM((HALF, D_MODEL), jnp.int32)] * DEPTH
        + [pltpu.SemaphoreType.DMA] * (2 * DEPTH),
    )
    def k(table_hbm, idx_hbm, out_hbm, idx_v, *bufs_sems):
        def pack(ibuf, obuf):
            @plsc.parallel_loop(0, HALF, unroll=2)
            def _(r):
                for c in range(D_MODEL // NUM_LANES):
                    sl = pl.ds(c * NUM_LANES, NUM_LANES)
                    u = plsc.bitcast(ibuf.at[r, sl][...], jnp.uint32)
                    v = plsc.bitcast(ibuf.at[r + HALF, sl][...], jnp.uint32)
                    rnd = jnp.uint32(0x8000)
                    lo = lax.shift_right_logical(u + rnd, jnp.uint32(16))
                    hi = (v + rnd) & jnp.uint32(0xFFFF0000)
                    obuf.at[r, sl][...] = plsc.bitcast(hi | lo, jnp.int32)

        _sc_pipeline(n_chunks, table_hbm, idx_hbm, out_hbm, idx_v, bufs_sems,
                     pack, HALF, lambda wid: wid * (n_chunks * HALF))

    return k


def _build_sc_scale(n_chunks: int, total_rows: int, row_offset: int):
    mesh = plsc.VectorSubcoreMesh(core_axis_name="c", subcore_axis_name="s")

    @functools.partial(
        pl.kernel,
        mesh=mesh,
        compiler_params=_compiler_params(),
        out_type=jax.ShapeDtypeStruct((total_rows, D_MODEL), jnp.float32),
        scratch_types=[
            pltpu.VMEM((n_chunks * CHUNK,), jnp.int32),
        ]
        + [pltpu.VMEM((CHUNK, D_MODEL), jnp.float32)] * (2 * DEPTH)
        + [pltpu.SemaphoreType.DMA] * (2 * DEPTH),
    )
    def k(table_hbm, idx_hbm, out_hbm, idx_v, *bufs_sems):
        def scale(ibuf, obuf):
            @plsc.parallel_loop(0, CHUNK, unroll=4)
            def _(r):
                for c in range(D_MODEL // NUM_LANES):
                    sl = pl.ds(c * NUM_LANES, NUM_LANES)
                    obuf.at[r, sl][...] = ibuf.at[r, sl][...] * SCALE

        _sc_pipeline(n_chunks, table_hbm, idx_hbm, out_hbm, idx_v, bufs_sems,
                     scale, CHUNK,
                     lambda wid: row_offset + wid * (n_chunks * CHUNK))

    return k


def _tc_unpack_body(pk_ref, raw_ref, out_ref):
    del raw_ref
    for i in range(TC_SUB):
        w = pk_ref[i * HALF:(i + 1) * HALF, :]
        lo = jax.lax.bitcast_convert_type(lax.shift_left(w, 16), jnp.float32)
        hi = jax.lax.bitcast_convert_type(w & jnp.int32(-65536), jnp.float32)
        out_ref[i * CHUNK:i * CHUNK + HALF, :] = lo * SCALE
        out_ref[i * CHUNK + HALF:(i + 1) * CHUNK, :] = hi * SCALE


def _tc_unpack(pk, raw_out):
    n_pack_rows = pk.shape[0]
    total_rows = raw_out.shape[0]
    grid = (n_pack_rows // (TC_SUB * HALF),)
    return pl.pallas_call(
        _tc_unpack_body,
        grid=grid,
        in_specs=[
            pl.BlockSpec((TC_SUB * HALF, D_MODEL), lambda g: (g, 0)),
            pl.BlockSpec(memory_space=pl.ANY),
        ],
        out_specs=pl.BlockSpec((TC_SUB * CHUNK, D_MODEL), lambda g: (g, 0)),
        out_shape=jax.ShapeDtypeStruct((total_rows, D_MODEL), jnp.float32),
        input_output_aliases={1: 0},
    )(pk, raw_out)


def kernel(x, table):
    b, l = x.shape
    vocab, d = table.shape
    assert d == D_MODEL
    n_total = b * l
    n_half = n_total // 2
    assert n_total % (2 * NW * CHUNK) == 0
    nc = n_half // (NW * CHUNK)

    idx = x.reshape(n_total).astype(jnp.int32)
    idx_a = lax.slice(idx, (0,), (n_half,))
    idx_b = lax.slice(idx, (n_half,), (n_total,))

    pk_a = _build_sc_pack(nc)(table, idx_a)
    raw = _build_sc_scale(nc, n_total, n_half)(table, idx_b)
    out = _tc_unpack(pk_a, raw)
    return out.reshape(b, l, d)


# speedup vs baseline: 8.9036x; 1.7098x over previous
import functools
import math

import jax
import jax.numpy as jnp
from jax import lax
from jax.experimental import pallas as pl
from jax.experimental.pallas import tpu as pltpu
from jax.experimental.pallas import tpu_sc as plsc

D_MODEL = 128
SCALE = math.sqrt(float(D_MODEL))

NUM_CORES = 2
NUM_SUBCORES = 16
NUM_LANES = 16
NW = NUM_CORES * NUM_SUBCORES

CHUNK = 128


def _build_gather(n_chunks: int):
    assert n_chunks % 2 == 0
    mesh = plsc.VectorSubcoreMesh(core_axis_name="c", subcore_axis_name="s")
    total_rows = NW * n_chunks * CHUNK

    DEPTH = 3
    n_main = (n_chunks // DEPTH) * DEPTH

    @functools.partial(
        pl.kernel,
        mesh=mesh,
        out_type=jax.ShapeDtypeStruct((total_rows, D_MODEL), jnp.float32),
        scratch_types=[
            pltpu.VMEM((n_chunks * CHUNK,), jnp.int32),
        ]
        + [pltpu.VMEM((CHUNK, D_MODEL), jnp.float32)] * (2 * DEPTH)
        + [pltpu.SemaphoreType.DMA] * (2 * DEPTH),
    )
    def k(table_hbm, idx_hbm, out_hbm, idx_v, *bufs_sems):
        ibufs = bufs_sems[0:DEPTH]
        obufs = bufs_sems[DEPTH:2 * DEPTH]
        isems = bufs_sems[2 * DEPTH:3 * DEPTH]
        osems = bufs_sems[3 * DEPTH:4 * DEPTH]

        wid = lax.axis_index("s") * NUM_CORES + lax.axis_index("c")
        base = wid * (n_chunks * CHUNK)
        pltpu.sync_copy(idx_hbm.at[pl.ds(base, n_chunks * CHUNK)], idx_v)

        def start_gather(j, s):
            idx = idx_v.at[pl.ds(j * CHUNK, CHUNK)]
            pltpu.make_async_copy(table_hbm.at[idx], ibufs[s], isems[s]).start()

        def wait_gather(j, s):
            idx = idx_v.at[pl.ds(j * CHUNK, CHUNK)]
            pltpu.make_async_copy(table_hbm.at[idx], ibufs[s], isems[s]).wait()

        def start_store(j, s):
            dst = out_hbm.at[pl.ds(base + j * CHUNK, CHUNK)]
            pltpu.make_async_copy(obufs[s], dst, osems[s]).start()

        def wait_store(s):
            dst = out_hbm.at[pl.ds(base, CHUNK)]
            pltpu.make_async_copy(obufs[s], dst, osems[s]).wait()

        def scale(s):
            ibuf, obuf = ibufs[s], obufs[s]

            @plsc.parallel_loop(0, CHUNK, unroll=4)
            def _(r):
                for c in range(D_MODEL // NUM_LANES):
                    sl = pl.ds(c * NUM_LANES, NUM_LANES)
                    obuf.at[r, sl][...] = ibuf.at[r, sl][...] * SCALE

        for s in range(DEPTH):
            start_gather(s, s)

        @pl.loop(0, n_main, step=DEPTH)
        def _(j):
            for s in range(DEPTH):
                jj = j + s
                wait_gather(jj, s)

                @pl.when(jj >= DEPTH)
                def _():
                    wait_store(s)

                scale(s)
                if n_main + s >= n_chunks:
                    @pl.when(jj + DEPTH < n_chunks)
                    def _():
                        start_gather(jj + DEPTH, s)
                else:
                    start_gather(jj + DEPTH, s)
                start_store(jj, s)

        for j in range(n_main, n_chunks):
            s = j % DEPTH
            wait_gather(j, s)
            wait_store(s)
            scale(s)
            start_store(j, s)

        for s in range(DEPTH):
            wait_store(s)

    return k


def kernel(x, table):
    b, l = x.shape
    vocab, d = table.shape
    assert d == D_MODEL
    n_total = b * l
    assert n_total % (NW * CHUNK) == 0
    n_chunks = n_total // (NW * CHUNK)
    idx = x.reshape(n_total).astype(jnp.int32)
    out = _build_gather(n_chunks)(table, idx)
    return out.reshape(b, l, d)
